# baseline (speedup 1.0000x reference)
import functools

import jax
import jax.numpy as jnp
from jax import lax
from jax.experimental import pallas as pl
from jax.experimental.pallas import tpu as pltpu
from jax.experimental.pallas import tpu_sc as plsc

N = 10000
E = 320000
H = 128
G = 16

NC = 2
NS = 16
NW = NC * NS
EPW = E // NW
KC = 80
ICH = 2000
ROWS_PT = N // NS

NB = 1000
EB = 2000

_mesh = plsc.VectorSubcoreMesh(core_axis_name="c", subcore_axis_name="s")


def _silu(x):
    return x * jax.nn.sigmoid(x)


NBUF = 4
CPW = EPW // KC
MAINK = (CPW - 1) // NBUF
NBG = 5
MAINKG = CPW // NBG


@functools.partial(
    pl.kernel,
    mesh=_mesh,
    compiler_params=pltpu.CompilerParams(use_tc_tiling_on_sc=False),
    out_type=[
        jax.ShapeDtypeStruct((E, H), jnp.float32),
        jax.ShapeDtypeStruct((E, H), jnp.float32),
    ],
    scratch_types=[
        pltpu.VMEM((EPW,), jnp.int32),
        pltpu.VMEM((EPW,), jnp.int32),
    ]
    + [pltpu.VMEM((KC, H), jnp.float32) for _ in range(2 * NBG)]
    + [pltpu.SemaphoreType.DMA for _ in range(3 * NBG)],
)
def _gather_sc(h_hbm, send_hbm, rec_hbm, hs_hbm, hr_hbm, sidx, ridx, *scr):
    bufa = scr[0:NBG]
    bufb = scr[NBG:2 * NBG]
    sems = scr[2 * NBG:]
    sema = sems[0:NBG]
    semb = sems[NBG:2 * NBG]
    semw = sems[2 * NBG:3 * NBG]
    wid = lax.axis_index("s") * NC + lax.axis_index("c")
    base0 = wid * EPW
    pltpu.sync_copy(send_hbm.at[pl.ds(base0, EPW)], sidx)
    pltpu.sync_copy(rec_hbm.at[pl.ds(base0, EPW)], ridx)

    def fire(c, b):
        off = pl.multiple_of(c * KC, 8)
        pltpu.async_copy(h_hbm.at[sidx.at[pl.ds(off, KC)]], bufa[b], sema[b])
        pltpu.async_copy(h_hbm.at[ridx.at[pl.ds(off, KC)]], bufb[b], semb[b])

    def drain(c, b):
        off = pl.multiple_of(c * KC, 8)
        pltpu.make_async_copy(
            h_hbm.at[sidx.at[pl.ds(off, KC)]], bufa[b], sema[b]).wait()
        pltpu.make_async_copy(
            h_hbm.at[ridx.at[pl.ds(off, KC)]], bufb[b], semb[b]).wait()

    def wb_fire(c, b):
        row = pl.ds(base0 + c * KC, KC)
        pltpu.async_copy(bufa[b], hs_hbm.at[row], semw[b])
        pltpu.async_copy(bufb[b], hr_hbm.at[row], semw[b])

    def wb_drain(c, b):
        row = pl.ds(base0 + c * KC, KC)
        pltpu.make_async_copy(bufa[b], hs_hbm.at[row], semw[b]).wait()
        pltpu.make_async_copy(bufb[b], hr_hbm.at[row], semw[b]).wait()

    for b in range(NBG):
        fire(b, b)

    def ring(k, _):
        for b in range(NBG):
            c = k * NBG + b
            pb = (b + NBG - 1) % NBG

            def recycle():
                wb_drain(c - 1, pb)
                fire(c + NBG - 1, pb)

            if b == 0:
                @pl.when(k > 0)
                def _():
                    recycle()
            else:
                wb_drain(c - 1, pb)

                @pl.when(k < MAINKG - 1)
                def _():
                    fire(c + NBG - 1, pb)
            drain(c, b)
            wb_fire(c, b)
        return _

    lax.fori_loop(0, MAINKG, ring, None)
    wb_drain(CPW - 1, NBG - 1)

NB0 = 4
MAINK0 = (CPW - 1) // NB0


@functools.partial(
    pl.kernel,
    mesh=_mesh,
    compiler_params=pltpu.CompilerParams(use_tc_tiling_on_sc=False),
    out_type=[
        jax.ShapeDtypeStruct((E, H), jnp.float32),
        jax.ShapeDtypeStruct((E, H), jnp.float32),
        jax.ShapeDtypeStruct((E, 16), jnp.float32),
    ],
    scratch_types=[
        pltpu.VMEM((EPW,), jnp.int32),
        pltpu.VMEM((EPW,), jnp.int32),
    ]
    + [pltpu.VMEM((KC, H), jnp.float32) for _ in range(2 * NB0)]
    + [pltpu.VMEM((KC, 16), jnp.float32) for _ in range(3 * NB0)]
    + [pltpu.SemaphoreType.DMA for _ in range(5 * NB0)],
)
def _gather0_sc(h_hbm, p_hbm, send_hbm, rec_hbm, hs_hbm, hr_hbm, diff_hbm,
                sidx, ridx, *scr):
    bufa = scr[0:NB0]
    bufb = scr[NB0:2 * NB0]
    bufpa = scr[2 * NB0:3 * NB0]
    bufpb = scr[3 * NB0:4 * NB0]
    obufd = scr[4 * NB0:5 * NB0]
    sems = scr[5 * NB0:]
    sema = sems[0:NB0]
    semb = sems[NB0:2 * NB0]
    sempa = sems[2 * NB0:3 * NB0]
    sempb = sems[3 * NB0:4 * NB0]
    semw = sems[4 * NB0:5 * NB0]
    wid = lax.axis_index("s") * NC + lax.axis_index("c")
    base0 = wid * EPW
    pltpu.sync_copy(send_hbm.at[pl.ds(base0, EPW)], sidx)
    pltpu.sync_copy(rec_hbm.at[pl.ds(base0, EPW)], ridx)

    def fire(c, b):
        off = pl.multiple_of(c * KC, 8)
        si = sidx.at[pl.ds(off, KC)]
        ri = ridx.at[pl.ds(off, KC)]
        pltpu.async_copy(h_hbm.at[si], bufa[b], sema[b])
        pltpu.async_copy(h_hbm.at[ri], bufb[b], semb[b])
        pltpu.async_copy(p_hbm.at[si], bufpa[b], sempa[b])
        pltpu.async_copy(p_hbm.at[ri], bufpb[b], sempb[b])

    def drain(c, b):
        off = pl.multiple_of(c * KC, 8)
        si = sidx.at[pl.ds(off, KC)]
        ri = ridx.at[pl.ds(off, KC)]
        pltpu.make_async_copy(h_hbm.at[si], bufa[b], sema[b]).wait()
        pltpu.make_async_copy(h_hbm.at[ri], bufb[b], semb[b]).wait()
        pltpu.make_async_copy(p_hbm.at[si], bufpa[b], sempa[b]).wait()
        pltpu.make_async_copy(p_hbm.at[ri], bufpb[b], sempb[b]).wait()

    def combine(b):
        def subrow(r, _):
            obufd[b][r, pl.ds(0, 16)] = (
                bufpa[b][r, pl.ds(0, 16)] - bufpb[b][r, pl.ds(0, 16)]
            )
            return _

        lax.fori_loop(0, KC, subrow, None)

    def wb_fire(c, b):
        row = pl.ds(base0 + c * KC, KC)
        pltpu.async_copy(bufa[b], hs_hbm.at[row], semw[b])
        pltpu.async_copy(bufb[b], hr_hbm.at[row], semw[b])
        pltpu.async_copy(obufd[b], diff_hbm.at[row], semw[b])

    def wb_drain(c, b):
        row = pl.ds(base0 + c * KC, KC)
        pltpu.make_async_copy(bufa[b], hs_hbm.at[row], semw[b]).wait()
        pltpu.make_async_copy(bufb[b], hr_hbm.at[row], semw[b]).wait()
        pltpu.make_async_copy(obufd[b], diff_hbm.at[row], semw[b]).wait()

    for b in range(NB0):
        fire(b, b)

    def ring(k, _):
        for b in range(NB0):
            c = k * NB0 + b
            pb = (b + NB0 - 1) % NB0

            def recycle():
                wb_drain(c - 1, pb)
                fire(c + NB0 - 1, pb)

            if b == 0:
                @pl.when(k > 0)
                def _():
                    recycle()
            elif b == 1:
                wb_drain(c - 1, pb)
                fire(c + NB0 - 1, pb)
            else:
                wb_drain(c - 1, pb)

                @pl.when(k < MAINK0 - 1)
                def _():
                    fire(c + NB0 - 1, pb)
            drain(c, b)
            combine(b)
            wb_fire(c, b)
        return _

    lax.fori_loop(0, MAINK0, ring, None)

    cl = CPW - 1
    bl = cl % NB0
    wb_drain(cl - 1, (bl + NB0 - 1) % NB0)
    drain(cl, bl)
    combine(bl)
    wb_fire(cl, bl)
    wb_drain(cl, bl)


@functools.partial(
    pl.kernel,
    mesh=_mesh,
    compiler_params=pltpu.CompilerParams(use_tc_tiling_on_sc=False),
    out_type=jax.ShapeDtypeStruct((NC, N, H), jnp.float32),
    scratch_types=[
        pltpu.VMEM_SHARED((N, H), jnp.float32),
        pltpu.VMEM((CPW, KC), jnp.int32),
    ]
    + [pltpu.VMEM((KC, H), jnp.float32) for _ in range(NBUF)]
    + [pltpu.SemaphoreType.DMA for _ in range(NBUF)],
)
def _scatter_sc(m_hbm, rect_hbm, zeros_hbm, agg_hbm, shared, ridx2, *scr):
    mbuf = scr[0:NBUF]
    sems = scr[NBUF:2 * NBUF]
    core = lax.axis_index("c")
    sub = lax.axis_index("s")
    chunks_per_core = E // KC // NC

    def g_of(r):
        return core * chunks_per_core + r * NS + sub

    def fire(r, b):
        pltpu.async_copy(m_hbm.at[pl.ds(g_of(r) * KC, KC)], mbuf[b], sems[b])

    def drain(r, b):
        pltpu.make_async_copy(
            m_hbm.at[pl.ds(g_of(r) * KC, KC)], mbuf[b], sems[b]).wait()

    for b in range(NBUF):
        fire(b, b)
    pltpu.sync_copy(rect_hbm.at[core, sub], ridx2)
    pltpu.sync_copy(zeros_hbm, shared.at[pl.ds(sub * ROWS_PT, ROWS_PT)])
    plsc.subcore_barrier()

    def ring(k, _):
        for b in range(NBUF):
            r = k * NBUF + b
            drain(r, b)
            pltpu.sync_copy(mbuf[b], shared.at[ridx2.at[r]], add=True)
            plsc.subcore_barrier()

            @pl.when(k < MAINK - 1)
            def _():
                fire(r + NBUF, b)
        return _

    lax.fori_loop(0, MAINK, ring, None)
    cl = CPW - 1
    fire(cl, 0)
    drain(cl, 0)
    pltpu.sync_copy(mbuf[0], shared.at[ridx2.at[cl]], add=True)
    plsc.subcore_barrier()
    pltpu.sync_copy(
        shared.at[pl.ds(sub * ROWS_PT, ROWS_PT)],
        agg_hbm.at[core, pl.ds(sub * ROWS_PT, ROWS_PT)],
    )


def _embed_body(xc, w1, b1, w2, b2, o):
    t = _silu(jnp.dot(xc[...], w1[...], preferred_element_type=jnp.float32) + b1[...])
    o[...] = jnp.dot(t, w2[...], preferred_element_type=jnp.float32) + b2[...]


def _edge0_body(hs, hr, diff, w1, b1, w2, b2, om, od2):
    d = diff[...]
    d2v = jnp.sum(d * d, axis=1, keepdims=True)
    od2[...] = d2v
    dist = jnp.where(d2v > 0, jnp.sqrt(jnp.where(d2v > 0, d2v, 1.0)), 0.0)
    state = jnp.concatenate([hs[...], hr[...], dist], axis=1)
    t = _silu(jnp.dot(state, w1[...], preferred_element_type=jnp.float32) + b1[...])
    om[...] = _silu(jnp.dot(t, w2[...], preferred_element_type=jnp.float32) + b2[...])


def _edge_body(hs, hr, d2, w1, b1, w2, b2, om):
    d2v = d2[...]
    dist = jnp.where(d2v > 0, jnp.sqrt(jnp.where(d2v > 0, d2v, 1.0)), 0.0)
    state = jnp.concatenate([hs[...], hr[...], dist], axis=1)
    t = _silu(jnp.dot(state, w1[...], preferred_element_type=jnp.float32) + b1[...])
    om[...] = _silu(jnp.dot(t, w2[...], preferred_element_type=jnp.float32) + b2[...])


def _node_body(h, p0, p1, u1, ub1, u2, ub2, oh):
    hv = h[...]
    aggr = p0[0] + p1[0]
    cat = jnp.concatenate([hv, aggr], axis=1)
    t = _silu(jnp.dot(cat, u1[...], preferred_element_type=jnp.float32) + ub1[...])
    oh[...] = hv + jnp.dot(t, u2[...], preferred_element_type=jnp.float32) + ub2[...]


def _final_body(h, bt, pw1, pb1, pw2, pb2, rw1, rb1, rw2, rb2, o, acc):
    i = pl.program_id(0)
    t = _silu(jnp.dot(h[...], pw1[...], preferred_element_type=jnp.float32) + pb1[...])
    z = jnp.dot(t, pw2[...], preferred_element_type=jnp.float32) + pb2[...]
    btv = bt[...]
    part = jnp.concatenate(
        [jnp.sum(jnp.where(btv == g, z, 0.0), axis=0, keepdims=True)
         for g in range(G)], axis=0)

    @pl.when(i == 0)
    def _():
        acc[...] = part

    @pl.when(i > 0)
    def _():
        acc[...] = acc[...] + part

    @pl.when(i == pl.num_programs(0) - 1)
    def _():
        tp = _silu(jnp.dot(acc[...], rw1[...], preferred_element_type=jnp.float32)
                   + rb1[...])
        o[...] = jnp.dot(tp, rw2[...], preferred_element_type=jnp.float32) + rb2[...]


def _full(r, c):
    return pl.BlockSpec((r, c), lambda i: (0, 0))


def kernel(x, pos, pe, edge_index, batch, embed_W1, embed_b1, embed_W2, embed_b2,
           msg_W1, msg_b1, msg_W2, msg_b2, upd_W1, upd_b1, upd_W2, upd_b2,
           pre_W1, pre_b1, pre_W2, pre_b2, ro_W1, ro_b1, ro_W2, ro_b2):
    L = msg_W1.shape[0]
    send = edge_index[0]
    rec = edge_index[1]
    rec_t = rec.reshape(NC, CPW, NS, KC).transpose(0, 2, 1, 3)
    posq = jnp.pad(pos, ((0, 0), (0, 13)))
    xcat = jnp.concatenate([x, pe], axis=-1)
    zeros = jnp.zeros((ROWS_PT, H), jnp.float32)

    nrow = pl.BlockSpec((NB, H), lambda i: (i, 0))
    erow = pl.BlockSpec((EB, H), lambda i: (i, 0))
    h = pl.pallas_call(
        _embed_body,
        grid=(N // NB,),
        in_specs=[nrow, _full(H, H), _full(1, H), _full(H, H), _full(1, H)],
        out_specs=nrow,
        out_shape=jax.ShapeDtypeStruct((N, H), jnp.float32),
    )(xcat, embed_W1, embed_b1.reshape(1, H), embed_W2, embed_b2.reshape(1, H))

    d2c = None
    for l in range(L):
        if l == 0:
            hs, hr, diff = _gather0_sc(h, posq, send, rec)
            m, d2c = pl.pallas_call(
                _edge0_body,
                grid=(E // EB,),
                in_specs=[
                    erow, erow,
                    pl.BlockSpec((EB, 16), lambda i: (i, 0)),
                    _full(2 * H + 1, H), _full(1, H), _full(H, H), _full(1, H),
                ],
                out_specs=[erow, pl.BlockSpec((EB, 1), lambda i: (i, 0))],
                out_shape=[
                    jax.ShapeDtypeStruct((E, H), jnp.float32),
                    jax.ShapeDtypeStruct((E, 1), jnp.float32),
                ],
            )(hs, hr, diff, msg_W1[l], msg_b1[l].reshape(1, H),
              msg_W2[l], msg_b2[l].reshape(1, H))
        else:
            hs, hr = _gather_sc(h, send, rec)
            m = pl.pallas_call(
                _edge_body,
                grid=(E // EB,),
                in_specs=[
                    erow, erow,
                    pl.BlockSpec((EB, 1), lambda i: (i, 0)),
                    _full(2 * H + 1, H), _full(1, H), _full(H, H), _full(1, H),
                ],
                out_specs=erow,
                out_shape=jax.ShapeDtypeStruct((E, H), jnp.float32),
            )(hs, hr, d2c, msg_W1[l], msg_b1[l].reshape(1, H),
              msg_W2[l], msg_b2[l].reshape(1, H))

        agg = _scatter_sc(m, rec_t, zeros)

        h = pl.pallas_call(
            _node_body,
            grid=(N // NB,),
            in_specs=[
                nrow,
                pl.BlockSpec((1, NB, H), lambda i: (0, i, 0)),
                pl.BlockSpec((1, NB, H), lambda i: (1, i, 0)),
                _full(2 * H, H), _full(1, H), _full(H, H), _full(1, H),
            ],
            out_specs=nrow,
            out_shape=jax.ShapeDtypeStruct((N, H), jnp.float32),
        )(h, agg, agg, upd_W1[l], upd_b1[l].reshape(1, H),
          upd_W2[l], upd_b2[l].reshape(1, H))

    out = pl.pallas_call(
        _final_body,
        grid=(N // NB,),
        in_specs=[
            pl.BlockSpec((NB, H), lambda i: (i, 0)),
            pl.BlockSpec((NB, 1), lambda i: (i, 0)),
            _full(H, H), _full(1, H), _full(H, H), _full(1, H),
            _full(H, H), _full(1, H), _full(H, 1), _full(1, 1),
        ],
        out_specs=_full(G, 1),
        out_shape=jax.ShapeDtypeStruct((G, 1), jnp.float32),
        scratch_shapes=[pltpu.VMEM((G, H), jnp.float32)],
    )(h, batch.reshape(N, 1), pre_W1, pre_b1.reshape(1, H), pre_W2,
      pre_b2.reshape(1, H), ro_W1, ro_b1.reshape(1, H), ro_W2,
      ro_b2.reshape(1, 1))

    return out.reshape(G)
